# baseline (speedup 1.0000x reference)
import functools

import jax
import jax.numpy as jnp
from jax import lax
from jax.experimental import pallas as pl
from jax.experimental.pallas import tpu as pltpu
from jax.experimental.pallas import tpu_sc as plsc

_DECAY = 1e-4
_EPS = 0.03

_NC = 2
_NS = 16
_NW = _NC * _NS


def _sc_gather(user4, item4, nu4, ni4, users4, pos4, neg4):
  B = users4.shape[0]
  BK = neg4.shape[0]
  per_w = B // _NW
  negs_per_w = BK // _NW
  chunk = min(per_w, 512)
  n_chunks = negs_per_w // chunk

  mesh = plsc.VectorSubcoreMesh(core_axis_name="c", subcore_axis_name="s",
                                num_cores=_NC)
  f32 = jnp.float32

  @functools.partial(
      pl.kernel,
      out_type=[
          jax.ShapeDtypeStruct((B, 128), f32),
          jax.ShapeDtypeStruct((B, 128), f32),
          jax.ShapeDtypeStruct((B, 128), f32),
          jax.ShapeDtypeStruct((B, 128), f32),
          jax.ShapeDtypeStruct((BK, 128), f32),
      ],
      mesh=mesh,
      scratch_types=[
          pltpu.VMEM((chunk,), jnp.int32),
          pltpu.VMEM((chunk, 128), f32),
          pltpu.SemaphoreType.DMA,
      ],
  )
  def gather_kernel(user_hbm, item_hbm, nu_hbm, ni_hbm, users_hbm, pos_hbm,
                    neg_hbm, u_out, pos_out, nu_out, ni_out, neg_out,
                    idx_v, rows_v, sem):
    wid = lax.axis_index("s") * _NC + lax.axis_index("c")
    base = wid * per_w
    pltpu.sync_copy(users_hbm.at[pl.ds(base, per_w)], idx_v)
    pltpu.async_copy(user_hbm.at[idx_v], rows_v, sem).wait()
    pltpu.sync_copy(rows_v, u_out.at[pl.ds(base, per_w)])
    pltpu.async_copy(nu_hbm.at[idx_v], rows_v, sem).wait()
    pltpu.sync_copy(rows_v, nu_out.at[pl.ds(base, per_w)])
    pltpu.sync_copy(pos_hbm.at[pl.ds(base, per_w)], idx_v)
    pltpu.async_copy(item_hbm.at[idx_v], rows_v, sem).wait()
    pltpu.sync_copy(rows_v, pos_out.at[pl.ds(base, per_w)])
    pltpu.async_copy(ni_hbm.at[idx_v], rows_v, sem).wait()
    pltpu.sync_copy(rows_v, ni_out.at[pl.ds(base, per_w)])
    nbase = wid * negs_per_w
    for t in range(n_chunks):
      off = nbase + t * chunk
      pltpu.sync_copy(neg_hbm.at[pl.ds(off, chunk)], idx_v)
      pltpu.async_copy(item_hbm.at[idx_v], rows_v, sem).wait()
      pltpu.sync_copy(rows_v, neg_out.at[pl.ds(off, chunk)])

  return gather_kernel(user4, item4, nu4, ni4, users4, pos4, neg4)


def _tc_score(u4, pos4, nu4, ni4, neg4, sel_u, sel_p, sel_n, B, K, D):
  Bb = min(512, B)
  grid = B // Bb
  f32 = jnp.float32

  def body(u_ref, pos_ref, nu_ref, ni_ref, neg_ref, su_ref, sp_ref, sn_ref,
           ypos_ref, yneg_ref, loss_ref, clu_ref, cli_ref):
    i = pl.program_id(0)

    def extract2(x4, sel):
      out = x4[:, 0:D]
      for k in range(1, 4):
        out = jnp.where(sel == k, x4[:, k * D:(k + 1) * D], out)
      return out

    def extract3(x4, sel):
      out = x4[:, :, 0:D]
      for k in range(1, 4):
        out = jnp.where(sel == k, x4[:, :, k * D:(k + 1) * D], out)
      return out

    u = extract2(u_ref[...], su_ref[...]) * 0.5
    pos = extract2(pos_ref[...], sp_ref[...]) * 0.5
    nu = extract2(nu_ref[...], su_ref[...]) * 0.5
    ni = extract2(ni_ref[...], sp_ref[...]) * 0.5
    neg = extract3(neg_ref[...], sn_ref[...][:, :, None]) * 0.5

    def inv_norm(x):
      n2 = jnp.sum(x * x, axis=-1, keepdims=True)
      return 1.0 / jnp.maximum(jnp.sqrt(n2), 1e-12)

    u_n = u * inv_norm(u)
    ypos_ref[...] = jnp.sum(pos * u_n, axis=-1, keepdims=True) * inv_norm(pos)
    neg_dot = jnp.sum(neg * u_n[:, None, :], axis=-1)
    neg_n2 = jnp.sum(neg * neg, axis=-1)
    yneg_ref[...] = neg_dot / jnp.maximum(jnp.sqrt(neg_n2), 1e-12)

    clu_ref[...] = u + jnp.sign(u) * (nu * inv_norm(nu)) * _EPS
    cli_ref[...] = pos + jnp.sign(pos) * (ni * inv_norm(ni)) * _EPS

    part = jnp.sum(u * u) + jnp.sum(pos * pos) + jnp.sum(neg * neg)

    @pl.when(i == 0)
    def _():
      loss_ref[...] = jnp.zeros((1, 1), jnp.float32)

    loss_ref[...] = loss_ref[...] + part

    @pl.when(i == pl.num_programs(0) - 1)
    def _():
      loss_ref[...] = loss_ref[...] * (_DECAY / (2.0 * B))

  return pl.pallas_call(
      body,
      grid=(grid,),
      in_specs=[
          pl.BlockSpec((Bb, 128), lambda i: (i, 0)),
          pl.BlockSpec((Bb, 128), lambda i: (i, 0)),
          pl.BlockSpec((Bb, 128), lambda i: (i, 0)),
          pl.BlockSpec((Bb, 128), lambda i: (i, 0)),
          pl.BlockSpec((Bb, K, 128), lambda i: (i, 0, 0)),
          pl.BlockSpec((Bb, 1), lambda i: (i, 0)),
          pl.BlockSpec((Bb, 1), lambda i: (i, 0)),
          pl.BlockSpec((Bb, K), lambda i: (i, 0)),
      ],
      out_specs=[
          pl.BlockSpec((Bb, 1), lambda i: (i, 0)),
          pl.BlockSpec((Bb, K), lambda i: (i, 0)),
          pl.BlockSpec((1, 1), lambda i: (0, 0)),
          pl.BlockSpec((Bb, D), lambda i: (i, 0)),
          pl.BlockSpec((Bb, D), lambda i: (i, 0)),
      ],
      out_shape=[
          jax.ShapeDtypeStruct((B, 1), f32),
          jax.ShapeDtypeStruct((B, K), f32),
          jax.ShapeDtypeStruct((1, 1), f32),
          jax.ShapeDtypeStruct((B, D), f32),
          jax.ShapeDtypeStruct((B, D), f32),
      ],
  )(u4, pos4, nu4, ni4, neg4, sel_u, sel_p, sel_n)


def kernel(user_embed, item_embed, noise_u, noise_i, users, pos_items,
           neg_items):
  B = users.shape[0]
  K = neg_items.shape[1]
  D = user_embed.shape[1]
  G = 128 // D

  users = users.astype(jnp.int32)
  pos_items = pos_items.astype(jnp.int32)
  neg_flat = neg_items.astype(jnp.int32).reshape(B * K)

  user4 = (user_embed * 2.0).reshape(-1, 128)
  item4 = (item_embed * 2.0).reshape(-1, 128)
  nu4 = (noise_u * 2.0).reshape(-1, 128)
  ni4 = (noise_i * 2.0).reshape(-1, 128)

  u4, pos4, nu4r, ni4r, neg4r = _sc_gather(
      user4, item4, nu4, ni4, users // G, pos_items // G, neg_flat // G)

  neg4r = neg4r.reshape(B, K, 128)
  ypos, yneg, loss, cl_u_e, cl_i_e = _tc_score(
      u4, pos4, nu4r, ni4r, neg4r,
      (users % G)[:, None], (pos_items % G)[:, None],
      (neg_flat % G).reshape(B, K), B, K, D)

  y_pred = jnp.concatenate([ypos, yneg], axis=1)
  return (y_pred, loss[0, 0], cl_u_e, cl_i_e)


# speedup vs baseline: 1.0311x; 1.0311x over previous
import functools

import jax
import jax.numpy as jnp
from jax import lax
from jax.experimental import pallas as pl
from jax.experimental.pallas import tpu as pltpu
from jax.experimental.pallas import tpu_sc as plsc

_DECAY = 1e-4
_EPS = 0.03

_NC = 2
_NS = 16
_NW = _NC * _NS


def _sc_mesh():
  return plsc.VectorSubcoreMesh(core_axis_name="c", subcore_axis_name="s",
                                num_cores=_NC)


def _sc_gather_items(item4, pos4, neg4):
  B = pos4.shape[0]
  BK = neg4.shape[0]
  per_w = B // _NW
  negs_per_w = BK // _NW
  chunk = min(per_w, 512)
  n_chunks = negs_per_w // chunk
  f32 = jnp.float32

  @functools.partial(
      pl.kernel,
      out_type=[
          jax.ShapeDtypeStruct((B, 128), f32),
          jax.ShapeDtypeStruct((BK, 128), f32),
      ],
      mesh=_sc_mesh(),
      scratch_types=[
          pltpu.VMEM((chunk,), jnp.int32),
          pltpu.VMEM((chunk, 128), f32),
          pltpu.SemaphoreType.DMA,
      ],
  )
  def gather_kernel(item_hbm, pos_hbm, neg_hbm, pos_out, neg_out,
                    idx_v, rows_v, sem):
    wid = lax.axis_index("s") * _NC + lax.axis_index("c")
    base = wid * per_w
    pltpu.sync_copy(pos_hbm.at[pl.ds(base, per_w)], idx_v)
    pltpu.async_copy(item_hbm.at[idx_v], rows_v, sem).wait()
    pltpu.sync_copy(rows_v, pos_out.at[pl.ds(base, per_w)])
    nbase = wid * negs_per_w
    for t in range(n_chunks):
      off = nbase + t * chunk
      pltpu.sync_copy(neg_hbm.at[pl.ds(off, chunk)], idx_v)
      pltpu.async_copy(item_hbm.at[idx_v], rows_v, sem).wait()
      pltpu.sync_copy(rows_v, neg_out.at[pl.ds(off, chunk)])

  return gather_kernel(item4, pos4, neg4)


def _sc_gather_users(user4, nu4, ni4, users4, pos4):
  B = users4.shape[0]
  per_w = B // _NW
  chunk = min(per_w, 512)
  f32 = jnp.float32

  @functools.partial(
      pl.kernel,
      out_type=[
          jax.ShapeDtypeStruct((B, 128), f32),
          jax.ShapeDtypeStruct((B, 128), f32),
          jax.ShapeDtypeStruct((B, 128), f32),
      ],
      mesh=_sc_mesh(),
      scratch_types=[
          pltpu.VMEM((chunk,), jnp.int32),
          pltpu.VMEM((chunk, 128), f32),
          pltpu.SemaphoreType.DMA,
      ],
  )
  def gather_kernel(user_hbm, nu_hbm, ni_hbm, users_hbm, pos_hbm,
                    u_out, nu_out, ni_out, idx_v, rows_v, sem):
    wid = lax.axis_index("s") * _NC + lax.axis_index("c")
    base = wid * per_w
    pltpu.sync_copy(users_hbm.at[pl.ds(base, per_w)], idx_v)
    pltpu.async_copy(user_hbm.at[idx_v], rows_v, sem).wait()
    pltpu.sync_copy(rows_v, u_out.at[pl.ds(base, per_w)])
    pltpu.async_copy(nu_hbm.at[idx_v], rows_v, sem).wait()
    pltpu.sync_copy(rows_v, nu_out.at[pl.ds(base, per_w)])
    pltpu.sync_copy(pos_hbm.at[pl.ds(base, per_w)], idx_v)
    pltpu.async_copy(ni_hbm.at[idx_v], rows_v, sem).wait()
    pltpu.sync_copy(rows_v, ni_out.at[pl.ds(base, per_w)])

  return gather_kernel(user4, nu4, ni4, users4, pos4)


def _tc_relayout3(a, b, c):
  N = a.shape[0]
  N4 = N // 4
  Rb4 = 2000
  grid = N4 // Rb4
  nblk = N4 // Rb4
  f32 = jnp.float32

  def body(a0, a1, a2, a3, b0, b1, b2, b3, c0, c1, c2, c3,
           ao_ref, bo_ref, co_ref):
    ao_ref[...] = jnp.concatenate(
        [a0[...], a1[...], a2[...], a3[...]], axis=1)
    bo_ref[...] = jnp.concatenate(
        [b0[...], b1[...], b2[...], b3[...]], axis=1)
    co_ref[...] = jnp.concatenate(
        [c0[...], c1[...], c2[...], c3[...]], axis=1)

  in_specs = []
  for _ in range(3):
    for k in range(4):
      in_specs.append(
          pl.BlockSpec((Rb4, 32), lambda i, k=k: (k * nblk + i, 0)))

  return pl.pallas_call(
      body,
      grid=(grid,),
      in_specs=in_specs,
      out_specs=[pl.BlockSpec((Rb4, 128), lambda i: (i, 0))] * 3,
      out_shape=[jax.ShapeDtypeStruct((N4, 128), f32)] * 3,
  )(a, a, a, a, b, b, b, b, c, c, c, c)


def _tc_score(u4, pos4, nu4, ni4, neg4, sel_u, sel_p, sel_pn, sel_n,
              B, K, D):
  Bb = min(512, B)
  grid = B // Bb
  f32 = jnp.float32

  def body(u_ref, pos_ref, nu_ref, ni_ref, neg_ref, su_ref, sp_ref,
           spn_ref, sn_ref, ypos_ref, yneg_ref, loss_ref, clu_ref,
           cli_ref):
    i = pl.program_id(0)

    def extract2(x4, sel):
      out = x4[:, 0:D]
      for k in range(1, 4):
        out = jnp.where(sel == k, x4[:, k * D:(k + 1) * D], out)
      return out

    def extract3(x4, sel):
      out = x4[:, :, 0:D]
      for k in range(1, 4):
        out = jnp.where(sel == k, x4[:, :, k * D:(k + 1) * D], out)
      return out

    u = extract2(u_ref[...], su_ref[...])
    pos = extract2(pos_ref[...], sp_ref[...])
    nu = extract2(nu_ref[...], su_ref[...])
    ni = extract2(ni_ref[...], spn_ref[...])
    neg = extract3(neg_ref[...], sn_ref[...][:, :, None])

    def inv_norm(x):
      n2 = jnp.sum(x * x, axis=-1, keepdims=True)
      return 1.0 / jnp.maximum(jnp.sqrt(n2), 1e-12)

    u_n = u * inv_norm(u)
    ypos_ref[...] = jnp.sum(pos * u_n, axis=-1, keepdims=True) * inv_norm(pos)
    neg_dot = jnp.sum(neg * u_n[:, None, :], axis=-1)
    neg_n2 = jnp.sum(neg * neg, axis=-1)
    yneg_ref[...] = neg_dot / jnp.maximum(jnp.sqrt(neg_n2), 1e-12)

    clu_ref[...] = u + jnp.sign(u) * (nu * inv_norm(nu)) * _EPS
    cli_ref[...] = pos + jnp.sign(pos) * (ni * inv_norm(ni)) * _EPS

    part = jnp.sum(u * u) + jnp.sum(pos * pos) + jnp.sum(neg * neg)

    @pl.when(i == 0)
    def _():
      loss_ref[...] = jnp.zeros((1, 1), jnp.float32)

    loss_ref[...] = loss_ref[...] + part

    @pl.when(i == pl.num_programs(0) - 1)
    def _():
      loss_ref[...] = loss_ref[...] * (_DECAY / (2.0 * B))

  return pl.pallas_call(
      body,
      grid=(grid,),
      in_specs=[
          pl.BlockSpec((Bb, 128), lambda i: (i, 0)),
          pl.BlockSpec((Bb, 128), lambda i: (i, 0)),
          pl.BlockSpec((Bb, 128), lambda i: (i, 0)),
          pl.BlockSpec((Bb, 128), lambda i: (i, 0)),
          pl.BlockSpec((Bb, K, 128), lambda i: (i, 0, 0)),
          pl.BlockSpec((Bb, 1), lambda i: (i, 0)),
          pl.BlockSpec((Bb, 1), lambda i: (i, 0)),
          pl.BlockSpec((Bb, 1), lambda i: (i, 0)),
          pl.BlockSpec((Bb, K), lambda i: (i, 0)),
      ],
      out_specs=[
          pl.BlockSpec((Bb, 1), lambda i: (i, 0)),
          pl.BlockSpec((Bb, K), lambda i: (i, 0)),
          pl.BlockSpec((1, 1), lambda i: (0, 0)),
          pl.BlockSpec((Bb, D), lambda i: (i, 0)),
          pl.BlockSpec((Bb, D), lambda i: (i, 0)),
      ],
      out_shape=[
          jax.ShapeDtypeStruct((B, 1), f32),
          jax.ShapeDtypeStruct((B, K), f32),
          jax.ShapeDtypeStruct((1, 1), f32),
          jax.ShapeDtypeStruct((B, D), f32),
          jax.ShapeDtypeStruct((B, D), f32),
      ],
  )(u4, pos4, nu4, ni4, neg4, sel_u, sel_p, sel_pn, sel_n)


def kernel(user_embed, item_embed, noise_u, noise_i, users, pos_items,
           neg_items):
  B = users.shape[0]
  K = neg_items.shape[1]
  D = user_embed.shape[1]
  G = 128 // D

  users = users.astype(jnp.int32)
  pos_items = pos_items.astype(jnp.int32)
  neg_flat = neg_items.astype(jnp.int32).reshape(B * K)

  N4 = user_embed.shape[0] // 4

  item4 = item_embed.reshape(-1, 128)
  pos4, neg4r = _sc_gather_items(item4, pos_items // G, neg_flat // G)

  user4, nu4, ni4 = _tc_relayout3(user_embed, noise_u, noise_i)
  u4, nu4r, ni4r = _sc_gather_users(user4, nu4, ni4, users % N4,
                                    pos_items % N4)

  neg4r = neg4r.reshape(B, K, 128)
  ypos, yneg, loss, cl_u_e, cl_i_e = _tc_score(
      u4, pos4, nu4r, ni4r, neg4r,
      (users // N4)[:, None], (pos_items % G)[:, None],
      (pos_items // N4)[:, None],
      (neg_flat % G).reshape(B, K), B, K, D)

  y_pred = jnp.concatenate([ypos, yneg], axis=1)
  return (y_pred, loss[0, 0], cl_u_e, cl_i_e)
